# Initial kernel scaffold; baseline (speedup 1.0000x reference)
import jax
import jax.numpy as jnp
from jax.experimental import pallas as pl


def kernel(xyz, new_xyz, features):
    raise NotImplementedError("write your pallas kernel here")


# speedup vs baseline: 5.3343x; 5.3343x over previous
import functools

import jax
import jax.numpy as jnp
from jax import lax
from jax.experimental import pallas as pl
from jax.experimental.pallas import tpu as pltpu
from jax.experimental.pallas import tpu_sc as plsc

B, N, M, NS, C = 8, 4096, 1024, 32, 128
RADIUS = 0.12
R2 = RADIUS * RADIUS

NC, NSUB, L = 2, 16, 16
NW = NC * NSUB
MW = M // NW
KC = 8
OUTC = 3 + C


def _body(pxyz_hbm, q_hbm, feat_hbm, out_hbm,
          pxyz_v, q_v, idx_v, frows_v, obuf_v, oxyz_v, sem):
    wid = lax.axis_index("s") * NC + lax.axis_index("c")
    lanes = lax.iota(jnp.int32, L)
    zeros16 = jnp.zeros((L,), jnp.int32)

    def per_batch(b, _):
        pltpu.sync_copy(pxyz_hbm.at[b], pxyz_v)
        pltpu.sync_copy(q_hbm.at[b, wid], q_v)

        def per_query(j, _):
            jNS = j * NS
            qx = plsc.load_gather(q_v, [jnp.full((L,), 3 * j, jnp.int32)])
            qy = plsc.load_gather(q_v, [jnp.full((L,), 3 * j + 1, jnp.int32)])
            qz = plsc.load_gather(q_v, [jnp.full((L,), 3 * j + 2, jnp.int32)])
            idx_v[pl.ds(jNS, L)] = zeros16
            idx_v[pl.ds(jNS + L, L)] = zeros16

            def scan_chunk(k, cnt):
                base = k * L
                px = pxyz_v[pl.ds(base, L)]
                py = pxyz_v[pl.ds(N + base, L)]
                pz = pxyz_v[pl.ds(2 * N + base, L)]
                dx = qx - px
                dy = qy - py
                dz = qz - pz
                d2 = dx * dx + dy * dy + dz * dz
                mask = d2 < R2
                rank = cnt + plsc.cumsum(mask.astype(jnp.int32)) - 1
                wmask = mask & (rank < NS)
                plsc.store_scatter(idx_v, [jNS + rank], lanes + base,
                                   mask=wmask)
                return cnt + plsc.all_reduce_population_count(mask)

            cnt = lax.fori_loop(0, N // L, scan_chunk, zeros16)

            first = plsc.load_gather(idx_v, [jnp.full((L,), jNS, jnp.int32)])
            for h in range(NS // L):
                cur = idx_v[pl.ds(jNS + h * L, L)]
                keep = (lanes + h * L) < cnt
                sel = jnp.where(keep, cur, first)
                idx_v[pl.ds(jNS + h * L, L)] = sel
                for d in range(3):
                    g = plsc.load_gather(pxyz_v, [sel + d * N])
                    qd = (qx, qy, qz)[d]
                    oxyz_v[d, pl.ds(jNS + h * L, L)] = g - qd
            return 0

        lax.fori_loop(0, MW, per_query, 0)
        pltpu.sync_copy(oxyz_v,
                        out_hbm.at[b, pl.ds(0, 3), pl.ds(wid * MW * NS, MW * NS)])

        def per_chunk(cc, _):
            pltpu.sync_copy(feat_hbm.at[b, pl.ds(cc * KC, KC), :], frows_v)
            for c in range(KC):
                cvec = jnp.full((L,), c, jnp.int32)
                def gath(t, _):
                    iv = idx_v[pl.ds(t * L, L)]
                    g = plsc.load_gather(frows_v, [cvec, iv])
                    obuf_v[c, pl.ds(t * L, L)] = g
                    return 0
                lax.fori_loop(0, MW * NS // L, gath, 0)
            pltpu.sync_copy(
                obuf_v,
                out_hbm.at[b, pl.ds(3 + cc * KC, KC),
                           pl.ds(wid * MW * NS, MW * NS)])
            return 0

        lax.fori_loop(0, C // KC, per_chunk, 0)
        return 0

    lax.fori_loop(0, B, per_batch, 0)


@functools.partial(
    pl.kernel,
    out_type=jax.ShapeDtypeStruct((B, OUTC, M * NS), jnp.float32),
    mesh=plsc.VectorSubcoreMesh(core_axis_name="c", subcore_axis_name="s"),
    scratch_types=[
        pltpu.VMEM((3 * N,), jnp.float32),
        pltpu.VMEM((MW * 3,), jnp.float32),
        pltpu.VMEM((MW * NS,), jnp.int32),
        pltpu.VMEM((KC, N), jnp.float32),
        pltpu.VMEM((KC, MW * NS), jnp.float32),
        pltpu.VMEM((3, MW * NS), jnp.float32),
        pltpu.SemaphoreType.DMA,
    ],
    compiler_params=pltpu.CompilerParams(use_tc_tiling_on_sc=False,
                                         needs_layout_passes=False),
)
def _qg_kernel(pxyz_hbm, q_hbm, feat_hbm, out_hbm, *scratch):
    _body(pxyz_hbm, q_hbm, feat_hbm, out_hbm, *scratch)


def kernel(xyz, new_xyz, features):
    pxyz = jnp.transpose(xyz, (0, 2, 1)).reshape(B, 3 * N)
    q = new_xyz.reshape(B, NW, MW * 3)
    out = _qg_kernel(pxyz, q, features)
    return out.reshape(B, OUTC, M, NS)


# speedup vs baseline: 7.0830x; 1.3278x over previous
import functools

import jax
import jax.numpy as jnp
from jax import lax
from jax.experimental import pallas as pl
from jax.experimental.pallas import tpu as pltpu
from jax.experimental.pallas import tpu_sc as plsc

B, N, M, NS, C = 8, 4096, 1024, 32, 128
RADIUS = 0.12
R2 = RADIUS * RADIUS

NC, NSUB, L = 2, 16, 16
NW = NC * NSUB
MW = M // NW
NG = MW // L
KC = 8
NCH = C // KC
PU = 8
OUTC = 3 + C


def _ball_query_group(pxyz_v, q_v, idx_v, grp, lanes):
    qsel = (grp * L + lanes) * 3
    qx = plsc.load_gather(q_v, [qsel])
    qy = plsc.load_gather(q_v, [qsel + 1])
    qz = plsc.load_gather(q_v, [qsel + 2])
    qoff = (grp * L + lanes) * NS
    plsc.store_scatter(idx_v, [qoff], jnp.zeros((L,), jnp.int32))

    def step(k, cnt):
        base = k * PU
        for j in range(PU):
            p = base + j
            pvec = jnp.full((L,), p, jnp.int32)
            px = plsc.load_gather(pxyz_v, [pvec])
            py = plsc.load_gather(pxyz_v, [pvec + N])
            pz = plsc.load_gather(pxyz_v, [pvec + 2 * N])
            dx = qx - px
            dy = qy - py
            dz = qz - pz
            d2 = dx * dx + dy * dy + dz * dz
            mask = d2 < R2
            wmask = mask & (cnt < NS)
            plsc.store_scatter(idx_v, [qoff + cnt], pvec, mask=wmask)
            cnt = cnt + mask.astype(jnp.int32)
        return cnt

    cnt = lax.fori_loop(0, N // PU, step, jnp.zeros((L,), jnp.int32))

    first = plsc.load_gather(idx_v, [qoff])
    for s in range(1, NS):
        cur = plsc.load_gather(idx_v, [qoff + s])
        sel = jnp.where(cnt > s, cur, first)
        plsc.store_scatter(idx_v, [qoff + s], sel)


def _body(pxyz_hbm, q_hbm, feat_hbm, out_hbm,
          pxyz_v, q_v, idx_v, frows_v, obuf_v, oxyz_v, sems):
    wid = lax.axis_index("s") * NC + lax.axis_index("c")
    lanes = lax.iota(jnp.int32, L)
    obase = wid * MW * NS

    def per_batch(b, _):
        pltpu.sync_copy(pxyz_hbm.at[b], pxyz_v)
        pltpu.sync_copy(q_hbm.at[b, wid], q_v)

        for grp in range(NG):
            _ball_query_group(pxyz_v, q_v, idx_v, grp, lanes)

        def xyz_gather(t, _):
            iv = idx_v[pl.ds(t * L, L)]
            mv = lax.shift_right_logical(t * L + lanes, 5)
            for d in range(3):
                g = plsc.load_gather(pxyz_v, [iv + d * N])
                qd = plsc.load_gather(q_v, [mv * 3 + d])
                oxyz_v[d, pl.ds(t * L, L)] = g - qd
            return 0

        lax.fori_loop(0, MW * NS // L, xyz_gather, 0)
        pltpu.sync_copy(oxyz_v, out_hbm.at[b, pl.ds(0, 3), pl.ds(obase, MW * NS)])

        def fin_copy(cc, slot):
            return pltpu.make_async_copy(
                feat_hbm.at[b, pl.ds(cc * KC, KC), :], frows_v.at[slot],
                sems.at[slot])

        def fout_copy(cc, slot):
            return pltpu.make_async_copy(
                obuf_v.at[slot],
                out_hbm.at[b, pl.ds(3 + cc * KC, KC), pl.ds(obase, MW * NS)],
                sems.at[2 + slot])

        fin_copy(0, 0).start()
        for cc in range(NCH):
            slot = cc % 2
            fin_copy(cc, slot).wait()
            if cc + 1 < NCH:
                fin_copy(cc + 1, 1 - slot).start()
            if cc >= 2:
                fout_copy(cc - 2, slot).wait()

            def gath(t, _):
                iv = idx_v[pl.ds(t * L, L)]
                for c in range(KC):
                    g = plsc.load_gather(frows_v,
                                         [jnp.full((L,), slot, jnp.int32),
                                          jnp.full((L,), c, jnp.int32), iv])
                    obuf_v[slot, c, pl.ds(t * L, L)] = g
                return 0

            lax.fori_loop(0, MW * NS // L, gath, 0)
            fout_copy(cc, slot).start()
        fout_copy(NCH - 2, NCH % 2).wait()
        fout_copy(NCH - 1, (NCH - 1) % 2).wait()
        return 0

    lax.fori_loop(0, B, per_batch, 0)


@functools.partial(
    pl.kernel,
    out_type=jax.ShapeDtypeStruct((B, OUTC, M * NS), jnp.float32),
    mesh=plsc.VectorSubcoreMesh(core_axis_name="c", subcore_axis_name="s"),
    scratch_types=[
        pltpu.VMEM((3 * N,), jnp.float32),
        pltpu.VMEM((MW * 3,), jnp.float32),
        pltpu.VMEM((MW * NS,), jnp.int32),
        pltpu.VMEM((2, KC, N), jnp.float32),
        pltpu.VMEM((2, KC, MW * NS), jnp.float32),
        pltpu.VMEM((3, MW * NS), jnp.float32),
        pltpu.SemaphoreType.DMA((4,)),
    ],
    compiler_params=pltpu.CompilerParams(use_tc_tiling_on_sc=False,
                                         needs_layout_passes=False),
)
def _qg_kernel(pxyz_hbm, q_hbm, feat_hbm, out_hbm, *scratch):
    _body(pxyz_hbm, q_hbm, feat_hbm, out_hbm, *scratch)


def kernel(xyz, new_xyz, features):
    pxyz = jnp.transpose(xyz, (0, 2, 1)).reshape(B, 3 * N)
    q = new_xyz.reshape(B, NW, MW * 3)
    out = _qg_kernel(pxyz, q, features)
    return out.reshape(B, OUTC, M, NS)


# speedup vs baseline: 9.7303x; 1.3737x over previous
import functools

import jax
import jax.numpy as jnp
from jax import lax
from jax.experimental import pallas as pl
from jax.experimental.pallas import tpu as pltpu
from jax.experimental.pallas import tpu_sc as plsc

B, N, M, NS, C = 8, 4096, 1024, 32, 128
RADIUS = 0.12
R2 = RADIUS * RADIUS

NC, NSUB, L = 2, 16, 16
NW = NC * NSUB
MW = M // NW
NG = MW // L
KC = 8
TC_ROWS = 136
NSLAB = 17
PU = 8
OUTC = 3 + C


def _ball_query_group(pxyz_v, q_v, idx_v, grp, lanes):
    zeros = jnp.zeros((L,), jnp.int32)
    qsel = (grp * L + lanes) * 3
    qx = plsc.load_gather(q_v, [zeros, qsel])
    qy = plsc.load_gather(q_v, [zeros, qsel + 1])
    qz = plsc.load_gather(q_v, [zeros, qsel + 2])
    qoff = (grp * L + lanes) * NS
    plsc.store_scatter(idx_v, [qoff], zeros)

    def step(k, cnt):
        base = k * PU
        for j in range(PU):
            p = base + j
            pvec = jnp.full((L,), p, jnp.int32)
            px = plsc.load_gather(pxyz_v, [zeros, pvec])
            py = plsc.load_gather(pxyz_v, [zeros + 1, pvec])
            pz = plsc.load_gather(pxyz_v, [zeros + 2, pvec])
            dx = qx - px
            dy = qy - py
            dz = qz - pz
            d2 = dx * dx + dy * dy + dz * dz
            mask = d2 < R2
            wmask = mask & (cnt < NS)
            plsc.store_scatter(idx_v, [qoff + cnt], pvec, mask=wmask)
            cnt = cnt + mask.astype(jnp.int32)
        return cnt

    cnt = lax.fori_loop(0, N // PU, step, jnp.zeros((L,), jnp.int32))

    first = plsc.load_gather(idx_v, [qoff])
    for s in range(1, NS):
        cur = plsc.load_gather(idx_v, [qoff + s])
        sel = jnp.where(cnt > s, cur, first)
        plsc.store_scatter(idx_v, [qoff + s], sel)


def _body(tab_hbm, q_hbm, out_hbm, pxyz_v, q_v, idx_v, frows_v, obuf_v, sems):
    wid = lax.axis_index("s") * NC + lax.axis_index("c")
    lanes = lax.iota(jnp.int32, L)
    obase = wid * MW * NS

    def per_batch(b, _):
        pltpu.sync_copy(tab_hbm.at[b, pl.ds(0, 3), :], pxyz_v)
        pltpu.sync_copy(q_hbm.at[b, wid], q_v)

        for grp in range(NG):
            _ball_query_group(pxyz_v, q_v, idx_v, grp, lanes)

        def fin_copy(s, slot):
            return pltpu.make_async_copy(
                tab_hbm.at[b, pl.ds(s * KC, KC), :], frows_v.at[slot],
                sems.at[slot])

        def fout_copy(s, slot):
            nrow = KC if s < NSLAB - 1 else OUTC - KC * (NSLAB - 1)
            return pltpu.make_async_copy(
                obuf_v.at[slot, pl.ds(0, nrow)],
                out_hbm.at[b, pl.ds(s * KC, nrow), pl.ds(obase, MW * NS)],
                sems.at[2 + slot])

        fin_copy(0, 0).start()
        for s in range(NSLAB):
            slot = s % 2
            fin_copy(s, slot).wait()
            if s + 1 < NSLAB:
                fin_copy(s + 1, 1 - slot).start()
            if s >= 2:
                fout_copy(s - 2, slot).wait()

            def gath(t, _):
                iv = idx_v[pl.ds(t * L, L)]
                if s == 0:
                    mv3 = lax.shift_right_logical(t * L + lanes, 5) * 3
                for c in range(KC):
                    g = plsc.load_gather(frows_v,
                                         [jnp.full((L,), slot, jnp.int32),
                                          jnp.full((L,), c, jnp.int32), iv])
                    if s == 0 and c < 3:
                        qd = plsc.load_gather(
                            q_v, [jnp.zeros((L,), jnp.int32), mv3 + c])
                        g = g - qd
                    obuf_v[slot, c, pl.ds(t * L, L)] = g
                return 0

            lax.fori_loop(0, MW * NS // L, gath, 0)
            fout_copy(s, slot).start()
        fout_copy(NSLAB - 2, (NSLAB - 2) % 2).wait()
        fout_copy(NSLAB - 1, (NSLAB - 1) % 2).wait()
        return 0

    lax.fori_loop(0, B, per_batch, 0)


@functools.partial(
    pl.kernel,
    out_type=jax.ShapeDtypeStruct((B, OUTC, M * NS), jnp.float32),
    mesh=plsc.VectorSubcoreMesh(core_axis_name="c", subcore_axis_name="s"),
    scratch_types=[
        pltpu.VMEM((3, N), jnp.float32),
        pltpu.VMEM((1, MW * 3), jnp.float32),
        pltpu.VMEM((MW * NS,), jnp.int32),
        pltpu.VMEM((2, KC, N), jnp.float32),
        pltpu.VMEM((2, KC, MW * NS), jnp.float32),
        pltpu.SemaphoreType.DMA((4,)),
    ],
    compiler_params=pltpu.CompilerParams(needs_layout_passes=False),
)
def _qg_kernel(tab_hbm, q_hbm, out_hbm, *scratch):
    _body(tab_hbm, q_hbm, out_hbm, *scratch)


def kernel(xyz, new_xyz, features):
    xyz_t = jnp.transpose(xyz, (0, 2, 1))
    pad = jnp.zeros((B, TC_ROWS - 3 - C, N), jnp.float32)
    tab = jnp.concatenate([xyz_t, features, pad], axis=1)
    q = new_xyz.reshape(B, NW, 1, MW * 3)
    out = _qg_kernel(tab, q)
    return out.reshape(B, OUTC, M, NS)
